# baseline (speedup 1.0000x reference)
import functools

import jax
import jax.numpy as jnp
from jax import lax
from jax.experimental import pallas as pl
from jax.experimental.pallas import tpu as pltpu
from jax.experimental.pallas import tpu_sc as plsc

L = 16
K = 16


@functools.lru_cache(maxsize=None)
def _build(B, S, VOCAB, D, NUM_EXP):
    info = plsc.get_sparse_core_info()
    NC, NS = info.num_cores, info.num_subcores
    NW = NC * NS
    N = B * S
    assert N % (NW * L) == 0
    C = N // NW
    NG = C // L
    NROW = NG + 1
    PADN = NROW * L

    mesh = plsc.VectorSubcoreMesh(core_axis_name="c", subcore_axis_name="s")

    @functools.partial(
        pl.kernel,
        out_type=jax.ShapeDtypeStruct((N, D), jnp.float32),
        mesh=mesh,
        scratch_types=[
            pltpu.VMEM((C,), jnp.int32),
            pltpu.VMEM((C,), jnp.float32),
            pltpu.VMEM((C,), jnp.float32),
            pltpu.VMEM((PADN + L,), jnp.int32),
            pltpu.VMEM((PADN + L,), jnp.int32),
            pltpu.VMEM((PADN + L,), jnp.int32),
            pltpu.VMEM((PADN + L,), jnp.int32),
            pltpu.VMEM((NROW, L), jnp.int32),
            pltpu.VMEM((NROW, L), jnp.int32),
            pltpu.VMEM((K, D), jnp.float32),
            pltpu.VMEM((K, D), jnp.float32),
            pltpu.SemaphoreType.DMA,
            pltpu.SemaphoreType.DMA,
        ],
        compiler_params=pltpu.CompilerParams(needs_layout_passes=False),
    )
    def emb(tok_hbm, val_hbm, msk_hbm, wt_hbm, et_hbm, out_hbm,
            tok_v, val_v, msk_v, uidx, upos1, midx, mpos1, upos, mpos,
            wbuf, ebuf, sem_w, sem_e):
        cid = lax.axis_index("c")
        sid = lax.axis_index("s")
        wid = sid * NC + cid
        base = wid * C

        pltpu.sync_copy(tok_hbm.at[pl.ds(base, C)], tok_v)
        pltpu.sync_copy(val_hbm.at[pl.ds(base, C)], val_v)
        pltpu.sync_copy(msk_hbm.at[pl.ds(base, C)], msk_v)

        cu = jnp.int32(0)
        cm = jnp.int32(0)
        for j in range(NG):
            tv = tok_v[pl.ds(j * L, L)]
            vv = val_v[pl.ds(j * L, L)]
            mv = msk_v[pl.ds(j * L, L)]
            vi = vv.astype(jnp.int32)
            one = jnp.ones((L,), jnp.int32)
            zero = jnp.zeros((L,), jnp.int32)
            eid = one
            for t in (10, 100, 1000, 10000, 100000, 1000000,
                      10000000, 100000000):
                eid = eid + jnp.where(vi >= t, one, zero)
            eid = jnp.minimum(eid, jnp.int32(NUM_EXP - 1))
            m = mv != 0.0
            um = jnp.logical_not(m)
            p = base + j * L + lax.iota(jnp.int32, L)
            lane = lax.iota(jnp.int32, L)
            mi = jnp.where(m, one, zero)
            csm = mi
            gdn = lax.GatherDimensionNumbers(
                offset_dims=(), collapsed_slice_dims=(0,),
                start_index_map=(0,))
            for sh in (1, 2, 4, 8):
                shifted = lax.gather(
                    csm, jnp.maximum(lane - sh, 0)[:, None],
                    dimension_numbers=gdn, slice_sizes=(1,),
                    mode=lax.GatherScatterMode.PROMISE_IN_BOUNDS)
                csm = csm + jnp.where(lane >= sh, shifted, zero)
            cnt = csm[L - 1]
            trash = jnp.int32(PADN) + lane
            dstm = jnp.where(m, cm + csm - 1, trash)
            plsc.store_scatter(midx, [dstm], eid)
            plsc.store_scatter(mpos1, [dstm], p)
            csu = (lane + 1) - csm
            dstu = jnp.where(um, cu + csu - 1, trash)
            plsc.store_scatter(uidx, [dstu], tv)
            plsc.store_scatter(upos1, [dstu], p)
            cm = cm + cnt
            cu = cu + (jnp.int32(L) - cnt)

        z16 = jnp.zeros((L,), jnp.int32)
        uidx[pl.ds(cu, L)] = plsc.load_gather(uidx, [z16])
        upos1[pl.ds(cu, L)] = plsc.load_gather(upos1, [z16])
        midx[pl.ds(cm, L)] = plsc.load_gather(midx, [z16])
        mpos1[pl.ds(cm, L)] = plsc.load_gather(mpos1, [z16])

        for r in range(NROW):
            upos[r, :] = upos1[pl.ds(r * L, L)]
            mpos[r, :] = mpos1[pl.ds(r * L, L)]

        nwc = (cu + (K - 1)) // K
        nec = (cm + (K - 1)) // K

        def wbody(c, carry):
            pltpu.async_copy(
                wt_hbm.at[uidx.at[pl.ds(c * K, K)]], wbuf, sem_w).wait()
            pltpu.async_copy(wbuf, out_hbm.at[upos.at[c]], sem_w).wait()
            return carry

        lax.fori_loop(0, nwc, wbody, jnp.int32(0))

        def ebody(c, carry):
            pltpu.async_copy(
                et_hbm.at[midx.at[pl.ds(c * K, K)]], ebuf, sem_e).wait()
            pltpu.async_copy(ebuf, out_hbm.at[mpos.at[c]], sem_e).wait()
            return carry

        lax.fori_loop(0, nec, ebody, jnp.int32(0))

    return emb


def kernel(token_ids, numeric_values, numeric_masks, word_table, exp_table):
    B, S = token_ids.shape
    VOCAB, D = word_table.shape
    NUM_EXP = exp_table.shape[0]
    N = B * S
    emb = _build(B, S, VOCAB, D, NUM_EXP)
    out = emb(
        token_ids.reshape(N),
        numeric_values.reshape(N),
        numeric_masks.reshape(N),
        word_table,
        exp_table,
    )
    return out.reshape(B, S, D)


# speedup vs baseline: 1.0125x; 1.0125x over previous
import functools

import jax
import jax.numpy as jnp
from jax import lax
from jax.experimental import pallas as pl
from jax.experimental.pallas import tpu as pltpu
from jax.experimental.pallas import tpu_sc as plsc

L = 16
K = 16
NBUF = 4


@functools.lru_cache(maxsize=None)
def _build(B, S, VOCAB, D, NUM_EXP):
    info = plsc.get_sparse_core_info()
    NC, NS = info.num_cores, info.num_subcores
    NW = NC * NS
    N = B * S
    assert N % (NW * L) == 0
    C = N // NW
    NG = C // L
    NROW = C // K + 1
    PADN = NROW * K

    mesh = plsc.VectorSubcoreMesh(core_axis_name="c", subcore_axis_name="s")

    @functools.partial(
        pl.kernel,
        out_type=jax.ShapeDtypeStruct((N, D), jnp.float32),
        mesh=mesh,
        scratch_types=[
            pltpu.VMEM((C,), jnp.int32),
            pltpu.VMEM((C,), jnp.float32),
            pltpu.VMEM((C,), jnp.float32),
            pltpu.VMEM((PADN + L,), jnp.int32),
            pltpu.VMEM((PADN + L,), jnp.int32),
            pltpu.VMEM((PADN + L,), jnp.int32),
            pltpu.VMEM((PADN + L,), jnp.int32),
            pltpu.VMEM((2 * NROW, L), jnp.int32),
            [pltpu.VMEM((K, D), jnp.float32) for _ in range(NBUF)],
            [pltpu.SemaphoreType.DMA for _ in range(NBUF)],
            [pltpu.SemaphoreType.DMA for _ in range(NBUF)],
        ],
        compiler_params=pltpu.CompilerParams(needs_layout_passes=False),
    )
    def emb(tok_hbm, val_hbm, msk_hbm, wt_hbm, et_hbm, out_hbm,
            tok_v, val_v, msk_v, uidx, upos1, midx, mpos1, cpos,
            bufs, gsems, ssems):
        cid = lax.axis_index("c")
        sid = lax.axis_index("s")
        wid = sid * NC + cid
        base = wid * C

        pltpu.sync_copy(tok_hbm.at[pl.ds(base, C)], tok_v)
        pltpu.sync_copy(val_hbm.at[pl.ds(base, C)], val_v)
        pltpu.sync_copy(msk_hbm.at[pl.ds(base, C)], msk_v)

        one = jnp.ones((L,), jnp.int32)
        zero = jnp.zeros((L,), jnp.int32)
        lane = lax.iota(jnp.int32, L)
        gdn = lax.GatherDimensionNumbers(
            offset_dims=(), collapsed_slice_dims=(0,), start_index_map=(0,))

        cu = jnp.int32(0)
        cm = jnp.int32(0)
        for j in range(NG):
            tv = tok_v[pl.ds(j * L, L)]
            vv = val_v[pl.ds(j * L, L)]
            mv = msk_v[pl.ds(j * L, L)]
            vi = vv.astype(jnp.int32)
            eid = one
            for t in (10, 100, 1000, 10000, 100000, 1000000,
                      10000000, 100000000):
                eid = eid + jnp.where(vi >= t, one, zero)
            eid = jnp.minimum(eid, jnp.int32(NUM_EXP - 1))
            m = mv != 0.0
            um = jnp.logical_not(m)
            p = base + j * L + lane
            mi = jnp.where(m, one, zero)
            csm = mi
            for sh in (1, 2, 4, 8):
                shifted = lax.gather(
                    csm, jnp.maximum(lane - sh, 0)[:, None],
                    dimension_numbers=gdn, slice_sizes=(1,),
                    mode=lax.GatherScatterMode.PROMISE_IN_BOUNDS)
                csm = csm + jnp.where(lane >= sh, shifted, zero)
            cnt = csm[L - 1]
            trash = jnp.int32(PADN) + lane
            dstm = jnp.where(m, cm + csm - 1, trash)
            plsc.store_scatter(midx, [dstm], eid)
            plsc.store_scatter(mpos1, [dstm], p)
            csu = (lane + 1) - csm
            dstu = jnp.where(um, cu + csu - 1, trash)
            plsc.store_scatter(uidx, [dstu], tv)
            plsc.store_scatter(upos1, [dstu], p)
            cm = cm + cnt
            cu = cu + (jnp.int32(L) - cnt)

        z16 = jnp.zeros((L,), jnp.int32)
        uidx[pl.ds(cu, L)] = plsc.load_gather(uidx, [z16])
        upos1[pl.ds(cu, L)] = plsc.load_gather(upos1, [z16])
        midx[pl.ds(cm, L)] = plsc.load_gather(midx, [z16])
        mpos1[pl.ds(cm, L)] = plsc.load_gather(mpos1, [z16])

        for r in range(NROW):
            cpos[r, :] = upos1[pl.ds(r * L, L)]
            cpos[NROW + r, :] = mpos1[pl.ds(r * L, L)]

        nwc = (cu + (K - 1)) // K
        nec = (cm + (K - 1)) // K
        ntc = nwc + nec

        def issue_gather(c, b):
            @pl.when(c < nwc)
            def _():
                pltpu.async_copy(
                    wt_hbm.at[uidx.at[pl.ds(c * K, K)]], bufs[b], gsems[b])

            @pl.when(jnp.logical_and(c >= nwc, c < ntc))
            def _():
                pltpu.async_copy(
                    et_hbm.at[midx.at[pl.ds((c - nwc) * K, K)]],
                    bufs[b], gsems[b])

        def wait_gather(b):
            pltpu.make_async_copy(
                wt_hbm.at[pl.ds(0, K)], bufs[b], gsems[b]).wait()

        def issue_scatter(c, b):
            r2 = jnp.where(c < nwc, c, NROW + (c - nwc))
            pltpu.async_copy(bufs[b], out_hbm.at[cpos.at[r2]], ssems[b])

        def wait_scatter(b):
            pltpu.make_async_copy(
                bufs[b], out_hbm.at[pl.ds(0, K)], ssems[b]).wait()

        for b in range(NBUF):
            issue_gather(jnp.int32(b), b)

        rounds = (ntc + NBUF - 1) // NBUF

        def rbody(r, carry):
            for b in range(NBUF):
                c = r * NBUF + b

                def stage(c=c, b=b):
                    wait_gather(b)
                    issue_scatter(c, b)

                    def refill(c=c, b=b):
                        wait_scatter(b)
                        issue_gather(c + NBUF, b)

                    pl.when(c + NBUF < ntc)(refill)

                pl.when(c < ntc)(stage)
            return carry

        lax.fori_loop(0, rounds, rbody, jnp.int32(0))

        for b in range(NBUF):
            pl.when(jnp.int32(b) < ntc)(lambda b=b: wait_scatter(b))

    return emb


def kernel(token_ids, numeric_values, numeric_masks, word_table, exp_table):
    B, S = token_ids.shape
    VOCAB, D = word_table.shape
    NUM_EXP = exp_table.shape[0]
    N = B * S
    emb = _build(B, S, VOCAB, D, NUM_EXP)
    out = emb(
        token_ids.reshape(N),
        numeric_values.reshape(N),
        numeric_masks.reshape(N),
        word_table,
        exp_table,
    )
    return out.reshape(B, S, D)
